# baseline (speedup 1.0000x reference)
import jax
import jax.numpy as jnp
from jax import lax
from jax.experimental import pallas as pl
from jax.experimental.pallas import tpu as pltpu
from jax.experimental.pallas import tpu_sc as plsc

B = 4096
L = 200
D = 64
NW = 32


def _build(batch, seq_len, d_model, num_workers, interpret=False):
    rpw = batch // num_workers

    lp = ((seq_len + 15) // 16) * 16
    nch = lp // 16
    nacc = d_model // 16

    def body(inputs_hbm, emb_hbm, w_hbm, out_hbm,
             idx_a, idx_b, rows_a, rows_b, wv_a, wv_b, out_v,
             sem_a, sem_b):
        if interpret:
            wid = 0
        else:
            wid = lax.axis_index("s") * 2 + lax.axis_index("c")
        base = pl.multiple_of(wid * rpw, 8)

        idx = [idx_a, idx_b]
        rows = [rows_a, rows_b]
        wv = [wv_a, wv_b]
        sem = [sem_a, sem_b]

        if lp > seq_len:
            for s in range(2):
                idx[s][pl.ds(lp - 16, 16)] = jnp.zeros((16,), jnp.int32)
                for l0 in range(seq_len, lp):
                    for c in range(nacc):
                        rows[s][l0, pl.ds(16 * c, 16)] = (
                            jnp.zeros((16,), jnp.float32))

        def _gidx(s):
            if interpret:
                return idx[s], rows[s], wv[s]
            return (idx[s].at[pl.ds(0, seq_len)],
                    rows[s].at[pl.ds(0, seq_len)],
                    wv[s].at[pl.ds(0, seq_len)])

        def fire(r, s):
            pltpu.sync_copy(inputs_hbm.at[base + r],
                            idx[s].at[pl.ds(0, seq_len)])
            ii, rr, ww = _gidx(s)
            pltpu.async_copy(emb_hbm.at[ii], rr, sem[s])
            pltpu.async_copy(w_hbm.at[ii], ww, sem[s])

        def drain(s):
            ii, rr, ww = _gidx(s)
            pltpu.make_async_copy(emb_hbm.at[ii], rr, sem[s]).wait()
            pltpu.make_async_copy(w_hbm.at[ii], ww, sem[s]).wait()

        def compute(r, s):
            accs = [jnp.zeros((16,), jnp.float32)] * (2 * nacc)
            for k in range(nch):
                iv = idx[s][pl.ds(16 * k, 16)]
                wvk = wv[s][pl.ds(16 * k, 16)]
                wmv = jnp.where(iv != 0, wvk,
                                jnp.zeros((16,), jnp.float32))
                for j in range(16):
                    w = wmv[j]
                    l = 16 * k + j
                    p = j % 2
                    for c in range(nacc):
                        accs[2 * c + p] = (
                            accs[2 * c + p]
                            + rows[s][l, pl.ds(16 * c, 16)] * w)
            for c in range(nacc):
                out_v[r, pl.ds(16 * c, 16)] = accs[2 * c] + accs[2 * c + 1]

        fire(0, 0)

        @pl.loop(0, rpw, step=2)
        def _rr(r0):
            for s in range(2):
                r = r0 + s
                nxt = r + 1

                @pl.when(nxt < rpw)
                def _():
                    fire(nxt, 1 - s)

                drain(s)
                compute(r, s)

        pltpu.sync_copy(out_v, out_hbm.at[pl.ds(base, rpw)])

    return pl.kernel(
        body,
        out_type=jax.ShapeDtypeStruct((batch, d_model), jnp.float32),
        mesh=plsc.VectorSubcoreMesh(core_axis_name="c",
                                    subcore_axis_name="s",
                                    num_cores=2, num_subcores=16),
        scratch_types=[
            pltpu.VMEM((lp,), jnp.int32),
            pltpu.VMEM((lp,), jnp.int32),
            pltpu.VMEM((lp, d_model), jnp.float32),
            pltpu.VMEM((lp, d_model), jnp.float32),
            pltpu.VMEM((lp,), jnp.float32),
            pltpu.VMEM((lp,), jnp.float32),
            pltpu.VMEM((rpw, d_model), jnp.float32),
            pltpu.SemaphoreType.DMA,
            pltpu.SemaphoreType.DMA,
        ],
        compiler_params=pltpu.CompilerParams(use_tc_tiling_on_sc=False),
        interpret=interpret,
    )


_sc_kernel = _build(B, L, D, NW)


def kernel(inputs, emb_table, weight_table):
    return _sc_kernel(inputs, emb_table, weight_table.reshape(-1))


# speedup vs baseline: 1.0052x; 1.0052x over previous
import jax
import jax.numpy as jnp
from jax import lax
from jax.experimental import pallas as pl
from jax.experimental.pallas import tpu as pltpu
from jax.experimental.pallas import tpu_sc as plsc

B = 4096
L = 200
D = 64
NW = 32
L_PAD = 256


def _build(batch, seq_len, d_model, num_workers, lstride=None,
           interpret=False):
    rpw = batch // num_workers
    d2 = 2 * d_model
    if lstride is None:
        lstride = seq_len

    lp = ((seq_len + 15) // 16) * 16
    nch = lp // 16
    nacc = d_model // 16

    def body(inputs_hbm, emb_hbm, w_hbm, out_hbm,
             idx_a, idx_b, idxh_a, idxh_b, rows_a, rows_b, wv_a, wv_b,
             out_v, sem_a, sem_b):
        if interpret:
            wid = 0
        else:
            wid = lax.axis_index("s") * 2 + lax.axis_index("c")
        base = pl.multiple_of(wid * rpw, 8)

        idx = [idx_a, idx_b]
        idxh = [idxh_a, idxh_b]
        rows = [rows_a, rows_b]
        wv = [wv_a, wv_b]
        sem = [sem_a, sem_b]

        if lp > seq_len:
            for s in range(2):
                idx[s][pl.ds(lp - 16, 16)] = jnp.zeros((16,), jnp.int32)

        def _gidx(s):
            if interpret:
                return idxh[s], idx[s], rows[s], wv[s]
            return (idxh[s].at[pl.ds(0, seq_len)],
                    idx[s].at[pl.ds(0, seq_len)],
                    rows[s].at[pl.ds(0, seq_len)],
                    wv[s].at[pl.ds(0, seq_len)])

        def fire(r, s):
            o = pl.multiple_of((base + r) * lstride, 8)
            pltpu.sync_copy(inputs_hbm.at[pl.ds(o, seq_len)],
                            idx[s].at[pl.ds(0, seq_len)])
            for k in range(nch):
                idxh[s][pl.ds(16 * k, 16)] = (
                    idx[s][pl.ds(16 * k, 16)] >> 1)
            ih, ii, rr, ww = _gidx(s)
            pltpu.async_copy(emb_hbm.at[ih], rr, sem[s])
            pltpu.async_copy(w_hbm.at[ii], ww, sem[s])

        def drain(s):
            ih, ii, rr, ww = _gidx(s)
            pltpu.make_async_copy(emb_hbm.at[ih], rr, sem[s]).wait()
            pltpu.make_async_copy(w_hbm.at[ii], ww, sem[s]).wait()

        def compute(r, s):
            accs = [jnp.zeros((16,), jnp.float32)] * (2 * nacc)
            for k in range(nch):
                iv = idx[s][pl.ds(16 * k, 16)]
                wvk = wv[s][pl.ds(16 * k, 16)]
                wmv = jnp.where(iv != 0, wvk,
                                jnp.zeros((16,), jnp.float32))
                ov = (iv & 1) * d_model
                for j in range(16):
                    l = 16 * k + j
                    if l >= seq_len:
                        break
                    w = wmv[j]
                    o = ov[j]
                    p = j % 2
                    for c in range(nacc):
                        accs[2 * c + p] = (
                            accs[2 * c + p]
                            + rows[s][l, pl.ds(o + 16 * c, 16)] * w)
            for c in range(nacc):
                out_v[r, pl.ds(16 * c, 16)] = accs[2 * c] + accs[2 * c + 1]

        fire(0, 0)

        @pl.loop(0, rpw, step=2)
        def _rr(r0):
            for s in range(2):
                r = r0 + s
                nxt = r + 1

                @pl.when(nxt < rpw)
                def _():
                    fire(nxt, 1 - s)

                drain(s)
                compute(r, s)

        pltpu.sync_copy(out_v, out_hbm.at[pl.ds(base, rpw)])

    return pl.kernel(
        body,
        out_type=jax.ShapeDtypeStruct((batch, d_model), jnp.float32),
        mesh=plsc.VectorSubcoreMesh(core_axis_name="c",
                                    subcore_axis_name="s",
                                    num_cores=2, num_subcores=16),
        scratch_types=[
            pltpu.VMEM((lp,), jnp.int32),
            pltpu.VMEM((lp,), jnp.int32),
            pltpu.VMEM((lp,), jnp.int32),
            pltpu.VMEM((lp,), jnp.int32),
            pltpu.VMEM((lp, d2), jnp.float32),
            pltpu.VMEM((lp, d2), jnp.float32),
            pltpu.VMEM((lp,), jnp.float32),
            pltpu.VMEM((lp,), jnp.float32),
            pltpu.VMEM((rpw, d_model), jnp.float32),
            pltpu.SemaphoreType.DMA,
            pltpu.SemaphoreType.DMA,
        ],
        compiler_params=pltpu.CompilerParams(use_tc_tiling_on_sc=False),
        interpret=interpret,
    )


_sc_kernel = _build(B, L, D, NW, lstride=L_PAD)


def kernel(inputs, emb_table, weight_table):
    ip = jnp.pad(inputs, ((0, 0), (0, L_PAD - L))).reshape(-1)
    emb_pairs = emb_table.reshape(emb_table.shape[0] // 2,
                                  2 * emb_table.shape[1])
    return _sc_kernel(ip, emb_pairs, weight_table.reshape(-1))
